# baseline (speedup 1.0000x reference)
import functools

import jax
import jax.numpy as jnp
from jax import lax
from jax.experimental import pallas as pl
from jax.experimental.pallas import tpu as pltpu
from jax.experimental.pallas import tpu_sc as plsc


@functools.cache
def _make_lookup(B: int, L: int, E: int):
    info = plsc.get_sparse_core_info()
    NC, NS = info.num_cores, info.num_subcores
    NW = NC * NS
    b_per_w = B // NW
    G = L // 8

    mesh = plsc.VectorSubcoreMesh(core_axis_name="c", subcore_axis_name="s")

    @functools.partial(
        pl.kernel,
        mesh=mesh,
        compiler_params=pltpu.CompilerParams(use_tc_tiling_on_sc=False),
        out_type=jax.ShapeDtypeStruct((2, B, G, 8, 128), jnp.float32),
        scratch_types=[
            pltpu.VMEM((L,), jnp.int32),
            pltpu.VMEM((L, E), jnp.float32),
            pltpu.SemaphoreType.DMA,
            pltpu.SemaphoreType.DMA,
        ],
    )
    def lookup(table, idx1, idx2, out, idx_v, rows_v, gsem, wsem):
        wid = lax.axis_index("s") * NC + lax.axis_index("c")
        base_b = wid * b_per_w

        def body(i, _):
            b = base_b + i
            off = b * L
            for s, idx in ((0, idx1), (1, idx2)):
                pltpu.sync_copy(idx.at[pl.ds(off, L)], idx_v)
                pltpu.async_copy(table.at[idx_v], rows_v, gsem).wait()
                descs = [
                    pltpu.async_copy(
                        rows_v.at[pl.ds(g * 8, 8)],
                        out.at[s, b, g, :, pl.ds(0, E)],
                        wsem,
                    )
                    for g in range(G)
                ]
                for d in descs:
                    d.wait()
            return ()

        lax.fori_loop(0, b_per_w, body, ())

    return lookup


def kernel(embeddings, input1, input2):
    b, l, nf = input1.shape
    e = embeddings.shape[1]
    idx1 = input1.reshape(b * l * nf)
    idx2 = input2.reshape(b * l * nf)
    padded = _make_lookup(b, l * nf, e)(embeddings, idx1, idx2)
    return padded[..., :e].reshape(2, b, l, nf * e)


# speedup vs baseline: 1.2099x; 1.2099x over previous
import functools

import jax
import jax.numpy as jnp
from jax import lax
from jax.experimental import pallas as pl
from jax.experimental.pallas import tpu as pltpu
from jax.experimental.pallas import tpu_sc as plsc


@functools.cache
def _make_lookup(BL: int, E: int, C: int):
    info = plsc.get_sparse_core_info()
    NC, NS = info.num_cores, info.num_subcores
    NW = NC * NS
    per_w = BL // NW
    n_chunks = per_w // C
    assert BL % NW == 0 and per_w % C == 0 and n_chunks % 2 == 0

    mesh = plsc.VectorSubcoreMesh(core_axis_name="c", subcore_axis_name="s")

    @functools.partial(
        pl.kernel,
        mesh=mesh,
        compiler_params=pltpu.CompilerParams(use_tc_tiling_on_sc=False),
        out_type=jax.ShapeDtypeStruct((2, BL, E), jnp.float32),
        scratch_types=[
            pltpu.VMEM((C,), jnp.int32),
            pltpu.VMEM((C,), jnp.int32),
            pltpu.VMEM((C, E), jnp.float32),
            pltpu.VMEM((C, E), jnp.float32),
            pltpu.SemaphoreType.DMA,
            pltpu.SemaphoreType.DMA,
            pltpu.SemaphoreType.DMA,
            pltpu.SemaphoreType.DMA,
        ],
    )
    def lookup(table, idx1, idx2, out, i0, i1, r0, r1, g0, g1, w0, w1):
        wid = lax.axis_index("s") * NC + lax.axis_index("c")
        base = wid * per_w

        def body(i, _):
            for s, idx in ((0, idx1), (1, idx2)):
                off0 = base + (2 * i) * C
                off1 = off0 + C
                pltpu.sync_copy(idx.at[pl.ds(off0, C)], i0)
                ga = pltpu.async_copy(table.at[i0], r0, g0)
                pltpu.sync_copy(idx.at[pl.ds(off1, C)], i1)
                gb = pltpu.async_copy(table.at[i1], r1, g1)
                ga.wait()
                wa = pltpu.async_copy(r0, out.at[s, pl.ds(off0, C)], w0)
                gb.wait()
                wb = pltpu.async_copy(r1, out.at[s, pl.ds(off1, C)], w1)
                wa.wait()
                wb.wait()
            return ()

        lax.fori_loop(0, n_chunks // 2, body, ())

    return lookup


def kernel(embeddings, input1, input2):
    b, l, nf = input1.shape
    e = embeddings.shape[1]
    BL = b * l * nf
    idx1 = input1.reshape(BL)
    idx2 = input2.reshape(BL)
    out = _make_lookup(BL, e, 1600)(embeddings, idx1, idx2)
    out128 = jax.lax.optimization_barrier(out.reshape(2 * BL * e // 128, 128))
    return out128.reshape(2, b, l, nf * e)
